# Initial kernel scaffold; baseline (speedup 1.0000x reference)
import jax
import jax.numpy as jnp
from jax.experimental import pallas as pl


def kernel(x, edge_index, edge_wt, batch, t1_Wq, t1_bq, t1_Wk, t1_bk, t1_Wv, t1_bv, t1_Ws, t1_bs, t0_Wq, t0_bq, t0_Wk, t0_bk, t0_Wv, t0_bv, t0_Ws, t0_bs, g_W, g_b, fc_W, fc_b):
    raise NotImplementedError("write your pallas kernel here")


# speedup vs baseline: 1.8715x; 1.8715x over previous
import functools

import jax
import jax.numpy as jnp
from jax import lax
from jax.experimental import pallas as pl
from jax.experimental.pallas import tpu as pltpu

N = 10000
E = 320000
D = 128

_ROWS = 1000


def _mm_body(x_ref, w_ref, b_ref, o_ref):
    o_ref[...] = (
        jnp.dot(x_ref[...], w_ref[...], preferred_element_type=jnp.float32)
        + b_ref[...]
    )


def _mm(x, w, b):
    n, d = x.shape
    k = w.shape[1]
    return pl.pallas_call(
        _mm_body,
        grid=(n // _ROWS,),
        in_specs=[
            pl.BlockSpec((_ROWS, d), lambda i: (i, 0)),
            pl.BlockSpec((d, k), lambda i: (0, 0)),
            pl.BlockSpec((1, k), lambda i: (0, 0)),
        ],
        out_specs=pl.BlockSpec((_ROWS, k), lambda i: (i, 0)),
        out_shape=jax.ShapeDtypeStruct((n, k), jnp.float32),
    )(x, w, b.reshape(1, k))


def _tconv(x, src, dst, Wq, bq, Wk, bk, Wv, bv, Ws, bs):
    n, d = x.shape
    w4 = jnp.concatenate([Wq, Wk, Wv, Ws], axis=1)
    b4 = jnp.concatenate([bq, bk, bv, bs])
    qkvr = _mm(x, w4, b4)
    q, k, v, r = (qkvr[:, 0:d], qkvr[:, d:2 * d], qkvr[:, 2 * d:3 * d],
                  qkvr[:, 3 * d:4 * d])
    alpha = jnp.sum(q[dst] * k[src], axis=-1) / jnp.sqrt(jnp.float32(d))
    gmax = jnp.max(alpha)
    ex = jnp.exp(alpha - gmax)
    den = jax.ops.segment_sum(ex, dst, num_segments=n)
    num = jax.ops.segment_sum(ex[:, None] * v[src], dst, num_segments=n)
    agg = num / (den[:, None] + 1e-16)
    return agg + r


def _gcn(x, src, dst, ew, W, b):
    n = x.shape[0]
    deg = jax.ops.segment_sum(ew, dst, num_segments=n) + 1.0
    dinv = lax.rsqrt(deg)
    norm = dinv[src] * ew * dinv[dst]
    xw = _mm(x, W, b * 0.0)
    out = jax.ops.segment_sum(norm[:, None] * xw[src], dst, num_segments=n)
    return out + xw / deg[:, None] + b


def kernel(x, edge_index, edge_wt, batch,
           t1_Wq, t1_bq, t1_Wk, t1_bk, t1_Wv, t1_bv, t1_Ws, t1_bs,
           t0_Wq, t0_bq, t0_Wk, t0_bk, t0_Wv, t0_bv, t0_Ws, t0_bs,
           g_W, g_b, fc_W, fc_b):
    src, dst = edge_index[0], edge_index[1]
    h = _tconv(x, src, dst, t1_Wq, t1_bq, t1_Wk, t1_bk, t1_Wv, t1_bv,
               t1_Ws, t1_bs)
    h = _gcn(h, src, dst, edge_wt, g_W, g_b)
    h = _tconv(h, src, dst, t0_Wq, t0_bq, t0_Wk, t0_bk, t0_Wv, t0_bv,
               t0_Ws, t0_bs)
    h = _tconv(h, src, dst, t1_Wq, t1_bq, t1_Wk, t1_bk, t1_Wv, t1_bv,
               t1_Ws, t1_bs)
    w_pad = jnp.pad(fc_W, ((0, 0), (0, 127)))
    b_pad = jnp.pad(fc_b, (0, 127))
    return _mm(h, w_pad, b_pad)[:, :1]


# speedup vs baseline: 9.3797x; 5.0118x over previous
import functools

import jax
import jax.numpy as jnp
from jax import lax
from jax.experimental import pallas as pl
from jax.experimental.pallas import tpu as pltpu
from jax.experimental.pallas import tpu_sc as plsc

N = 10000
E = 320000
D = 128

NC = 2
NS = 16
NW = NC * NS
CB = 128
NCHUNK = E // CB
NITER = -(-NCHUNK // NW)
NP = 10240
RPT = NP // NS
SCALE = 1.0 / float(D) ** 0.5
ROWS = 1000


def _mesh():
    return plsc.VectorSubcoreMesh(core_axis_name="c", subcore_axis_name="s")


@functools.partial(
    pl.kernel,
    out_type=(jax.ShapeDtypeStruct((E,), jnp.float32),
              jax.ShapeDtypeStruct((NW * 16,), jnp.float32)),
    mesh=_mesh(),
    scratch_types=[
        pltpu.VMEM((CB,), jnp.int32),
        pltpu.VMEM((CB,), jnp.int32),
        pltpu.VMEM((CB, D), jnp.float32),
        pltpu.VMEM((CB, D), jnp.float32),
        pltpu.VMEM((CB,), jnp.float32),
        pltpu.VMEM((16,), jnp.float32),
        pltpu.SemaphoreType.DMA,
        pltpu.SemaphoreType.DMA,
    ],
)
def _alpha_sc(q_hbm, k_hbm, src_hbm, dst_hbm, alpha_hbm, amax_hbm,
              srci, dsti, qv, kv, av, mxv, sem1, sem2):
    c = lax.axis_index("c")
    s = lax.axis_index("s")
    wid = s * NC + c
    mxv[...] = jnp.full((16,), -3e38, jnp.float32)
    i16 = lax.iota(jnp.int32, 16)
    sh8 = (i16 + 8) & 15
    sh4 = (i16 + 4) & 15
    sh2 = (i16 + 2) & 15
    sh1 = (i16 + 1) & 15

    def t_body(t, carry):
        chunk = t * NW + wid

        @pl.when(chunk < NCHUNK)
        def _():
            off = chunk * CB
            pltpu.sync_copy(dst_hbm.at[pl.ds(off, CB)], dsti)
            pltpu.sync_copy(src_hbm.at[pl.ds(off, CB)], srci)
            cp1 = pltpu.async_copy(q_hbm.at[dsti], qv, sem1)
            cp2 = pltpu.async_copy(k_hbm.at[srci], kv, sem2)
            cp1.wait()
            cp2.wait()

            def e_body(jj, c2):
                vec = jnp.zeros((16,), jnp.float32)
                for l in range(16):
                    i = jj * 16 + l
                    acc = qv[i, pl.ds(0, 16)] * kv[i, pl.ds(0, 16)]
                    for d8 in range(1, 8):
                        acc = acc + (qv[i, pl.ds(d8 * 16, 16)]
                                     * kv[i, pl.ds(d8 * 16, 16)])
                    acc = acc + acc[sh8]
                    acc = acc + acc[sh4]
                    acc = acc + acc[sh2]
                    acc = acc + acc[sh1]
                    vec = jnp.where(i16 == l, acc * SCALE, vec)
                av[pl.ds(jj * 16, 16)] = vec
                mxv[...] = jnp.maximum(mxv[...], vec)
                return c2

            lax.fori_loop(0, CB // 16, e_body, 0)
            pltpu.sync_copy(av, alpha_hbm.at[pl.ds(off, CB)])

        return carry

    lax.fori_loop(0, NITER, t_body, 0)
    pltpu.sync_copy(mxv, amax_hbm.at[pl.ds(wid * 16, 16)])


def _make_aggr(with_ew):
    scratch = [
        pltpu.VMEM((CB,), jnp.int32),
        pltpu.VMEM((CB,), jnp.int32),
        pltpu.VMEM((CB, D), jnp.float32),
        pltpu.VMEM((CB, D), jnp.float32),
        pltpu.VMEM((CB,), jnp.float32),
        pltpu.VMEM((CB,), jnp.float32),
        pltpu.VMEM((CB,), jnp.float32),
        pltpu.VMEM((16,), jnp.float32),
        pltpu.VMEM_SHARED((NP, D), jnp.float32),
        pltpu.VMEM_SHARED((NP,), jnp.float32),
        pltpu.VMEM_SHARED((NP,), jnp.float32),
        pltpu.SemaphoreType.DMA,
    ]
    outs = [jax.ShapeDtypeStruct((NC * NP, D), jnp.float32),
            jax.ShapeDtypeStruct((NC * NP,), jnp.float32)]
    if with_ew:
        outs.append(jax.ShapeDtypeStruct((NC * NP,), jnp.float32))

    @functools.partial(
        pl.kernel,
        out_type=tuple(outs),
        mesh=_mesh(),
        scratch_types=scratch,
    )
    def _aggr_sc(v_hbm, src_hbm, dst_hbm, alpha_hbm, gmax_hbm, ew_hbm,
                 zerosd_hbm, zeros1_hbm, *refs):
        if with_ew:
            (out_hbm, den_hbm, ewa_hbm,
             srci, dsti, vv, sv, av, exv, ewv, gv, acc, den, ewa,
             sem1) = refs
        else:
            (out_hbm, den_hbm,
             srci, dsti, vv, sv, av, exv, ewv, gv, acc, den, ewa,
             sem1) = refs
        c = lax.axis_index("c")
        s = lax.axis_index("s")
        wid = s * NC + c

        pltpu.sync_copy(zerosd_hbm.at[pl.ds(s * RPT, RPT)],
                        acc.at[pl.ds(s * RPT, RPT)])
        pltpu.sync_copy(zeros1_hbm.at[pl.ds(s * RPT, RPT)],
                        den.at[pl.ds(s * RPT, RPT)])
        if with_ew:
            pltpu.sync_copy(zeros1_hbm.at[pl.ds(s * RPT, RPT)],
                            ewa.at[pl.ds(s * RPT, RPT)])
        pltpu.sync_copy(gmax_hbm, gv)
        plsc.subcore_barrier()
        gvec = gv[...]

        def t_body(t, carry):
            chunk = t * NW + wid

            @pl.when(chunk < NCHUNK)
            def _():
                off = chunk * CB
                pltpu.sync_copy(dst_hbm.at[pl.ds(off, CB)], dsti)
                pltpu.sync_copy(src_hbm.at[pl.ds(off, CB)], srci)
                pltpu.sync_copy(alpha_hbm.at[pl.ds(off, CB)], av)
                cp1 = pltpu.async_copy(v_hbm.at[srci], vv, sem1)
                if with_ew:
                    pltpu.sync_copy(ew_hbm.at[pl.ds(off, CB)], ewv)
                    pltpu.sync_copy(ewv, ewa.at[dsti], add=True)

                def x_body(j, c2):
                    sl = pl.ds(j * 16, 16)
                    exv[sl] = jnp.exp(av[sl] - gvec)
                    return c2

                lax.fori_loop(0, CB // 16, x_body, 0)
                pltpu.sync_copy(exv, den.at[dsti], add=True)
                cp1.wait()

                def e_body(jj, c2):
                    exvec = exv[pl.ds(jj * 16, 16)]
                    for l in range(16):
                        i = jj * 16 + l
                        e = exvec[l]
                        for d8 in range(8):
                            sl = pl.ds(d8 * 16, 16)
                            sv[i, sl] = vv[i, sl] * e
                    return c2

                lax.fori_loop(0, CB // 16, e_body, 0)
                pltpu.sync_copy(sv, acc.at[dsti], add=True)

            return carry

        lax.fori_loop(0, NITER, t_body, 0)
        plsc.subcore_barrier()
        pltpu.sync_copy(acc.at[pl.ds(s * RPT, RPT)],
                        out_hbm.at[pl.ds(c * NP + s * RPT, RPT)])
        pltpu.sync_copy(den.at[pl.ds(s * RPT, RPT)],
                        den_hbm.at[pl.ds(c * NP + s * RPT, RPT)])
        if with_ew:
            pltpu.sync_copy(ewa.at[pl.ds(s * RPT, RPT)],
                            ewa_hbm.at[pl.ds(c * NP + s * RPT, RPT)])

    return _aggr_sc


_aggr_ew = _make_aggr(True)
_aggr_plain = _make_aggr(False)


@functools.partial(
    pl.kernel,
    out_type=jax.ShapeDtypeStruct((NC * NP, D), jnp.float32),
    mesh=_mesh(),
    scratch_types=[
        pltpu.VMEM((CB,), jnp.int32),
        pltpu.VMEM((CB,), jnp.int32),
        pltpu.VMEM((CB, D), jnp.float32),
        pltpu.VMEM((CB, D), jnp.float32),
        pltpu.VMEM((CB,), jnp.float32),
        pltpu.VMEM_SHARED((NP, D), jnp.float32),
        pltpu.SemaphoreType.DMA,
    ],
)
def _gcn_sc(xwp_hbm, src_hbm, dst_hbm, ew_hbm, zeros_hbm, out_hbm,
            srci, dsti, xwv, sv, ewv, acc, sem1):
    c = lax.axis_index("c")
    s = lax.axis_index("s")
    wid = s * NC + c

    pltpu.sync_copy(zeros_hbm.at[pl.ds(s * RPT, RPT)],
                    acc.at[pl.ds(s * RPT, RPT)])
    plsc.subcore_barrier()

    def t_body(t, carry):
        chunk = t * NW + wid

        @pl.when(chunk < NCHUNK)
        def _():
            off = chunk * CB
            pltpu.sync_copy(dst_hbm.at[pl.ds(off, CB)], dsti)
            pltpu.sync_copy(src_hbm.at[pl.ds(off, CB)], srci)
            pltpu.sync_copy(ew_hbm.at[pl.ds(off, CB)], ewv)
            cp1 = pltpu.async_copy(xwp_hbm.at[srci], xwv, sem1)
            cp1.wait()

            def e_body(jj, c2):
                evec = ewv[pl.ds(jj * 16, 16)]
                for l in range(16):
                    i = jj * 16 + l
                    nrm = evec[l]
                    for d8 in range(8):
                        sl = pl.ds(d8 * 16, 16)
                        sv[i, sl] = xwv[i, sl] * nrm
                return c2

            lax.fori_loop(0, CB // 16, e_body, 0)
            pltpu.sync_copy(sv, acc.at[dsti], add=True)

        return carry

    lax.fori_loop(0, NITER, t_body, 0)
    plsc.subcore_barrier()
    pltpu.sync_copy(acc.at[pl.ds(s * RPT, RPT)],
                    out_hbm.at[pl.ds(c * NP + s * RPT, RPT)])


def _mm4_body(x_ref, w_ref, b_ref, q_ref, k_ref, v_ref, r_ref):
    o = (jnp.dot(x_ref[...], w_ref[...], preferred_element_type=jnp.float32)
         + b_ref[...])
    q_ref[...] = o[:, 0:D]
    k_ref[...] = o[:, D:2 * D]
    v_ref[...] = o[:, 2 * D:3 * D]
    r_ref[...] = o[:, 3 * D:4 * D]


def _mm4(x, w4, b4):
    os = jax.ShapeDtypeStruct((N, D), jnp.float32)
    return pl.pallas_call(
        _mm4_body,
        grid=(N // ROWS,),
        in_specs=[
            pl.BlockSpec((ROWS, D), lambda i: (i, 0)),
            pl.BlockSpec((D, 4 * D), lambda i: (0, 0)),
            pl.BlockSpec((1, 4 * D), lambda i: (0, 0)),
        ],
        out_specs=[pl.BlockSpec((ROWS, D), lambda i: (i, 0))] * 4,
        out_shape=[os, os, os, os],
    )(x, w4, b4.reshape(1, 4 * D))


def _softmax_finish(acc_ref, dinv_ref, r_ref):
    return (acc_ref[0] + acc_ref[1]) * dinv_ref[...] + r_ref[...]


def _epA_mm_body(acc_ref, dinv_ref, r_ref, w_ref, b_ref, o_ref):
    h = _softmax_finish(acc_ref, dinv_ref, r_ref)
    o_ref[...] = (jnp.dot(h, w_ref[...], preferred_element_type=jnp.float32)
                  + b_ref[...])


def _epA_mm(acc, deninv, r, w, b):
    kk = w.shape[1]
    return pl.pallas_call(
        _epA_mm_body,
        grid=(N // ROWS,),
        in_specs=[
            pl.BlockSpec((2, ROWS, D), lambda i: (0, i, 0)),
            pl.BlockSpec((ROWS, 1), lambda i: (i, 0)),
            pl.BlockSpec((ROWS, D), lambda i: (i, 0)),
            pl.BlockSpec((D, kk), lambda i: (0, 0)),
            pl.BlockSpec((1, kk), lambda i: (0, 0)),
        ],
        out_specs=pl.BlockSpec((ROWS, kk), lambda i: (i, 0)),
        out_shape=jax.ShapeDtypeStruct((N, kk), jnp.float32),
    )(acc.reshape(NC, NP, D)[:, :N], deninv, r, w, b.reshape(1, kk))


def _epA_mm4_body(acc_ref, dinv_ref, r_ref, w_ref, b_ref,
                  q_ref, k_ref, v_ref, r4_ref):
    h = _softmax_finish(acc_ref, dinv_ref, r_ref)
    o = (jnp.dot(h, w_ref[...], preferred_element_type=jnp.float32)
         + b_ref[...])
    q_ref[...] = o[:, 0:D]
    k_ref[...] = o[:, D:2 * D]
    v_ref[...] = o[:, 2 * D:3 * D]
    r4_ref[...] = o[:, 3 * D:4 * D]


def _epA_mm4(acc, deninv, r, w4, b4):
    os = jax.ShapeDtypeStruct((N, D), jnp.float32)
    return pl.pallas_call(
        _epA_mm4_body,
        grid=(N // ROWS,),
        in_specs=[
            pl.BlockSpec((2, ROWS, D), lambda i: (0, i, 0)),
            pl.BlockSpec((ROWS, 1), lambda i: (i, 0)),
            pl.BlockSpec((ROWS, D), lambda i: (i, 0)),
            pl.BlockSpec((D, 4 * D), lambda i: (0, 0)),
            pl.BlockSpec((1, 4 * D), lambda i: (0, 0)),
        ],
        out_specs=[pl.BlockSpec((ROWS, D), lambda i: (i, 0))] * 4,
        out_shape=[os, os, os, os],
    )(acc.reshape(NC, NP, D)[:, :N], deninv, r, w4, b4.reshape(1, 4 * D))


def _epA_mm_gcn_body(acc_ref, deninv_ref, r_ref, w_ref, dinv_ref,
                     xw_ref, xwp_ref):
    h = _softmax_finish(acc_ref, deninv_ref, r_ref)
    xw = jnp.dot(h, w_ref[...], preferred_element_type=jnp.float32)
    xw_ref[...] = xw
    xwp_ref[...] = xw * dinv_ref[...]


def _epA_mm_gcn(acc, deninv, r, w, dinv):
    os = jax.ShapeDtypeStruct((N, D), jnp.float32)
    return pl.pallas_call(
        _epA_mm_gcn_body,
        grid=(N // ROWS,),
        in_specs=[
            pl.BlockSpec((2, ROWS, D), lambda i: (0, i, 0)),
            pl.BlockSpec((ROWS, 1), lambda i: (i, 0)),
            pl.BlockSpec((ROWS, D), lambda i: (i, 0)),
            pl.BlockSpec((D, D), lambda i: (0, 0)),
            pl.BlockSpec((ROWS, 1), lambda i: (i, 0)),
        ],
        out_specs=[pl.BlockSpec((ROWS, D), lambda i: (i, 0))] * 2,
        out_shape=[os, os],
    )(acc.reshape(NC, NP, D)[:, :N], deninv, r, w, dinv)


def _epG_mm4_body(acc_ref, xw_ref, dinv_ref, invdeg_ref, gb_ref, w_ref,
                  b_ref, q_ref, k_ref, v_ref, r4_ref):
    h = ((acc_ref[0] + acc_ref[1]) * dinv_ref[...]
         + xw_ref[...] * invdeg_ref[...] + gb_ref[...])
    o = (jnp.dot(h, w_ref[...], preferred_element_type=jnp.float32)
         + b_ref[...])
    q_ref[...] = o[:, 0:D]
    k_ref[...] = o[:, D:2 * D]
    v_ref[...] = o[:, 2 * D:3 * D]
    r4_ref[...] = o[:, 3 * D:4 * D]


def _epG_mm4(acc, xw, dinv, invdeg, g_b, w4, b4):
    os = jax.ShapeDtypeStruct((N, D), jnp.float32)
    return pl.pallas_call(
        _epG_mm4_body,
        grid=(N // ROWS,),
        in_specs=[
            pl.BlockSpec((2, ROWS, D), lambda i: (0, i, 0)),
            pl.BlockSpec((ROWS, D), lambda i: (i, 0)),
            pl.BlockSpec((ROWS, 1), lambda i: (i, 0)),
            pl.BlockSpec((ROWS, 1), lambda i: (i, 0)),
            pl.BlockSpec((1, D), lambda i: (0, 0)),
            pl.BlockSpec((D, 4 * D), lambda i: (0, 0)),
            pl.BlockSpec((1, 4 * D), lambda i: (0, 0)),
        ],
        out_specs=[pl.BlockSpec((ROWS, D), lambda i: (i, 0))] * 4,
        out_shape=[os, os, os, os],
    )(acc.reshape(NC, NP, D)[:, :N], xw, dinv, invdeg, g_b.reshape(1, D), w4,
      b4.reshape(1, 4 * D))


def _deninv(den):
    d2 = den.reshape(NC, NP)[:, :N]
    return (1.0 / (d2[0] + d2[1] + 1e-16))[:, None]


def kernel(x, edge_index, edge_wt, batch,
           t1_Wq, t1_bq, t1_Wk, t1_bk, t1_Wv, t1_bv, t1_Ws, t1_bs,
           t0_Wq, t0_bq, t0_Wk, t0_bk, t0_Wv, t0_bv, t0_Ws, t0_bs,
           g_W, g_b, fc_W, fc_b):
    src, dst = edge_index[0], edge_index[1]
    zerosD = jnp.zeros((NP, D), jnp.float32)
    zeros1 = jnp.zeros((NP,), jnp.float32)
    w4_t1 = jnp.concatenate([t1_Wq, t1_Wk, t1_Wv, t1_Ws], axis=1)
    b4_t1 = jnp.concatenate([t1_bq, t1_bk, t1_bv, t1_bs])
    w4_t0 = jnp.concatenate([t0_Wq, t0_Wk, t0_Wv, t0_Ws], axis=1)
    b4_t0 = jnp.concatenate([t0_bq, t0_bk, t0_bv, t0_bs])
    fcw_pad = jnp.pad(fc_W, ((0, 0), (0, D - 1)))
    fcb_pad = jnp.pad(fc_b, (0, D - 1))

    q, k, v, r = _mm4(x, w4_t1, b4_t1)
    alpha, amaxp = _alpha_sc(q, k, src, dst)
    gmax16 = jnp.full((16,), jnp.max(amaxp), jnp.float32)
    acc1, den1, ewa1 = _aggr_ew(v, src, dst, alpha, gmax16, edge_wt,
                                zerosD, zeros1)

    e2 = ewa1.reshape(NC, NP)[:, :N]
    deg = e2[0] + e2[1] + 1.0
    dinv = lax.rsqrt(deg)[:, None]
    invdeg = (1.0 / deg)[:, None]

    xw, xwp = _epA_mm_gcn(acc1, _deninv(den1), r, g_W, dinv)
    accC = _gcn_sc(xwp, src, dst, edge_wt, zerosD)

    q, k, v, r = _epG_mm4(accC, xw, dinv, invdeg, g_b, w4_t0, b4_t0)
    alpha, amaxp = _alpha_sc(q, k, src, dst)
    gmax16 = jnp.full((16,), jnp.max(amaxp), jnp.float32)
    acc2, den2 = _aggr_plain(v, src, dst, alpha, gmax16, edge_wt,
                             zerosD, zeros1)

    q, k, v, r = _epA_mm4(acc2, _deninv(den2), r, w4_t1, b4_t1)
    alpha, amaxp = _alpha_sc(q, k, src, dst)
    gmax16 = jnp.full((16,), jnp.max(amaxp), jnp.float32)
    acc3, den3 = _aggr_plain(v, src, dst, alpha, gmax16, edge_wt,
                             zerosD, zeros1)

    return _epA_mm(acc3, _deninv(den3), r, fcw_pad, fcb_pad)[:, 0:1]


# speedup vs baseline: 13.8145x; 1.4728x over previous
import functools

import jax
import jax.numpy as jnp
from jax import lax
from jax.experimental import pallas as pl
from jax.experimental.pallas import tpu as pltpu
from jax.experimental.pallas import tpu_sc as plsc

N = 10000
E = 320000
D = 128

NC = 2
NS = 16
NW = NC * NS
CB = 128
NCHUNK = E // CB
NITER = -(-NCHUNK // NW)
NPAIR = -(-NITER // 2)
NP = 10240
RPT = NP // NS
SCALE = 1.0 / float(D) ** 0.5
ROWS = 1000


def _mesh():
    return plsc.VectorSubcoreMesh(core_axis_name="c", subcore_axis_name="s")


@functools.partial(
    pl.kernel,
    out_type=(jax.ShapeDtypeStruct((E,), jnp.float32),
              jax.ShapeDtypeStruct((NW * 16,), jnp.float32)),
    mesh=_mesh(),
    scratch_types=[
        pltpu.VMEM((CB,), jnp.int32),
        pltpu.VMEM((CB,), jnp.int32),
        pltpu.VMEM((CB,), jnp.int32),
        pltpu.VMEM((CB,), jnp.int32),
        pltpu.VMEM((CB, D), jnp.float32),
        pltpu.VMEM((CB, D), jnp.float32),
        pltpu.VMEM((CB, D), jnp.float32),
        pltpu.VMEM((CB, D), jnp.float32),
        pltpu.VMEM((CB,), jnp.float32),
        pltpu.VMEM((16,), jnp.float32),
        pltpu.SemaphoreType.DMA,
        pltpu.SemaphoreType.DMA,
        pltpu.SemaphoreType.DMA,
        pltpu.SemaphoreType.DMA,
    ],
)
def _alpha_sc(q_hbm, k_hbm, src_hbm, dst_hbm, alpha_hbm, amax_hbm,
              srciA, dstiA, srciB, dstiB, qvA, kvA, qvB, kvB, av, mxv,
              semA1, semA2, semB1, semB2):
    c = lax.axis_index("c")
    s = lax.axis_index("s")
    wid = s * NC + c
    mxv[...] = jnp.full((16,), -3e38, jnp.float32)
    i16 = lax.iota(jnp.int32, 16)
    sh8 = (i16 + 8) & 15
    sh4 = (i16 + 4) & 15
    sh2 = (i16 + 2) & 15
    sh1 = (i16 + 1) & 15

    def issue(chunk, srci, dsti, qv, kv, s1, s2):
        off = chunk * CB
        pltpu.sync_copy(dst_hbm.at[pl.ds(off, CB)], dsti)
        pltpu.sync_copy(src_hbm.at[pl.ds(off, CB)], srci)
        pltpu.async_copy(q_hbm.at[dsti], qv, s1)
        pltpu.async_copy(k_hbm.at[srci], kv, s2)

    def process(chunk, srci, dsti, qv, kv, s1, s2):
        pltpu.make_async_copy(q_hbm.at[dsti], qv, s1).wait()
        pltpu.make_async_copy(k_hbm.at[srci], kv, s2).wait()

        def e_body(jj, c2):
            vec = jnp.zeros((16,), jnp.float32)
            for l in range(16):
                i = jj * 16 + l
                acc = qv[i, pl.ds(0, 16)] * kv[i, pl.ds(0, 16)]
                for d8 in range(1, 8):
                    acc = acc + (qv[i, pl.ds(d8 * 16, 16)]
                                 * kv[i, pl.ds(d8 * 16, 16)])
                acc = acc + acc[sh8]
                acc = acc + acc[sh4]
                acc = acc + acc[sh2]
                acc = acc + acc[sh1]
                vec = jnp.where(i16 == l, acc * SCALE, vec)
            av[pl.ds(jj * 16, 16)] = vec
            mxv[...] = jnp.maximum(mxv[...], vec)
            return c2

        lax.fori_loop(0, CB // 16, e_body, 0)
        pltpu.sync_copy(av, alpha_hbm.at[pl.ds(chunk * CB, CB)])

    issue(wid, srciA, dstiA, qvA, kvA, semA1, semA2)

    def t_body(tp, carry):
        c0 = (2 * tp) * NW + wid
        c1 = c0 + NW
        c2 = c1 + NW

        @pl.when(c1 < NCHUNK)
        def _():
            issue(c1, srciB, dstiB, qvB, kvB, semB1, semB2)

        @pl.when(c0 < NCHUNK)
        def _():
            process(c0, srciA, dstiA, qvA, kvA, semA1, semA2)

        @pl.when(c2 < NCHUNK)
        def _():
            issue(c2, srciA, dstiA, qvA, kvA, semA1, semA2)

        @pl.when(c1 < NCHUNK)
        def _():
            process(c1, srciB, dstiB, qvB, kvB, semB1, semB2)

        return carry

    lax.fori_loop(0, NPAIR, t_body, 0)
    pltpu.sync_copy(mxv, amax_hbm.at[pl.ds(wid * 16, 16)])


def _make_aggr(with_ew):
    scratch = [
        pltpu.VMEM((CB,), jnp.int32),
        pltpu.VMEM((CB,), jnp.int32),
        pltpu.VMEM((CB,), jnp.int32),
        pltpu.VMEM((CB,), jnp.int32),
        pltpu.VMEM((CB, D), jnp.float32),
        pltpu.VMEM((CB, D), jnp.float32),
        pltpu.VMEM((CB,), jnp.float32),
        pltpu.VMEM((CB,), jnp.float32),
        pltpu.VMEM((CB,), jnp.float32),
        pltpu.VMEM((CB,), jnp.float32),
        pltpu.VMEM((CB,), jnp.float32),
        pltpu.VMEM((16,), jnp.float32),
        pltpu.VMEM_SHARED((NP, D), jnp.float32),
        pltpu.VMEM_SHARED((NP,), jnp.float32),
        pltpu.VMEM_SHARED((NP,), jnp.float32),
        pltpu.SemaphoreType.DMA,
        pltpu.SemaphoreType.DMA,
    ]
    outs = [jax.ShapeDtypeStruct((NC * NP, D), jnp.float32),
            jax.ShapeDtypeStruct((NC * NP,), jnp.float32)]
    if with_ew:
        outs.append(jax.ShapeDtypeStruct((NC * NP,), jnp.float32))

    @functools.partial(
        pl.kernel,
        out_type=tuple(outs),
        mesh=_mesh(),
        scratch_types=scratch,
    )
    def _aggr_sc(v_hbm, src_hbm, dst_hbm, alpha_hbm, gmax_hbm, ew_hbm,
                 zerosd_hbm, zeros1_hbm, *refs):
        if with_ew:
            (out_hbm, den_hbm, ewa_hbm,
             srciA, dstiA, srciB, dstiB, vvA, vvB,
             avA, avB, ewvA, ewvB, exv, gv, acc, den, ewa,
             semA, semB) = refs
        else:
            (out_hbm, den_hbm,
             srciA, dstiA, srciB, dstiB, vvA, vvB,
             avA, avB, ewvA, ewvB, exv, gv, acc, den, ewa,
             semA, semB) = refs
        c = lax.axis_index("c")
        s = lax.axis_index("s")
        wid = s * NC + c

        pltpu.sync_copy(zerosd_hbm.at[pl.ds(s * RPT, RPT)],
                        acc.at[pl.ds(s * RPT, RPT)])
        pltpu.sync_copy(zeros1_hbm.at[pl.ds(s * RPT, RPT)],
                        den.at[pl.ds(s * RPT, RPT)])
        if with_ew:
            pltpu.sync_copy(zeros1_hbm.at[pl.ds(s * RPT, RPT)],
                            ewa.at[pl.ds(s * RPT, RPT)])
        pltpu.sync_copy(gmax_hbm, gv)
        plsc.subcore_barrier()
        gvec = gv[...]

        def issue(chunk, srci, dsti, vv, av, ewv, sem):
            off = chunk * CB
            pltpu.sync_copy(dst_hbm.at[pl.ds(off, CB)], dsti)
            pltpu.sync_copy(src_hbm.at[pl.ds(off, CB)], srci)
            pltpu.sync_copy(alpha_hbm.at[pl.ds(off, CB)], av)
            if with_ew:
                pltpu.sync_copy(ew_hbm.at[pl.ds(off, CB)], ewv)
            pltpu.async_copy(v_hbm.at[srci], vv, sem)

        def process(chunk, srci, dsti, vv, av, ewv, sem):
            if with_ew:
                pltpu.sync_copy(ewv, ewa.at[dsti], add=True)

            def x_body(j, c2):
                sl = pl.ds(j * 16, 16)
                exv[sl] = jnp.exp(av[sl] - gvec)
                return c2

            lax.fori_loop(0, CB // 16, x_body, 0)
            pltpu.sync_copy(exv, den.at[dsti], add=True)
            pltpu.make_async_copy(v_hbm.at[srci], vv, sem).wait()

            def e_body(jj, c2):
                exvec = exv[pl.ds(jj * 16, 16)]
                for l in range(16):
                    i = jj * 16 + l
                    e = exvec[l]
                    for d8 in range(8):
                        sl = pl.ds(d8 * 16, 16)
                        vv[i, sl] = vv[i, sl] * e
                return c2

            lax.fori_loop(0, CB // 16, e_body, 0)
            pltpu.sync_copy(vv, acc.at[dsti], add=True)

        issue(wid, srciA, dstiA, vvA, avA, ewvA, semA)

        def t_body(tp, carry):
            c0 = (2 * tp) * NW + wid
            c1 = c0 + NW
            c2 = c1 + NW

            @pl.when(c1 < NCHUNK)
            def _():
                issue(c1, srciB, dstiB, vvB, avB, ewvB, semB)

            @pl.when(c0 < NCHUNK)
            def _():
                process(c0, srciA, dstiA, vvA, avA, ewvA, semA)

            @pl.when(c2 < NCHUNK)
            def _():
                issue(c2, srciA, dstiA, vvA, avA, ewvA, semA)

            @pl.when(c1 < NCHUNK)
            def _():
                process(c1, srciB, dstiB, vvB, avB, ewvB, semB)

            return carry

        lax.fori_loop(0, NPAIR, t_body, 0)
        plsc.subcore_barrier()
        pltpu.sync_copy(acc.at[pl.ds(s * RPT, RPT)],
                        out_hbm.at[pl.ds(c * NP + s * RPT, RPT)])
        pltpu.sync_copy(den.at[pl.ds(s * RPT, RPT)],
                        den_hbm.at[pl.ds(c * NP + s * RPT, RPT)])
        if with_ew:
            pltpu.sync_copy(ewa.at[pl.ds(s * RPT, RPT)],
                            ewa_hbm.at[pl.ds(c * NP + s * RPT, RPT)])

    return _aggr_sc


_aggr_ew = _make_aggr(True)
_aggr_plain = _make_aggr(False)


@functools.partial(
    pl.kernel,
    out_type=jax.ShapeDtypeStruct((NC * NP, D), jnp.float32),
    mesh=_mesh(),
    scratch_types=[
        pltpu.VMEM((CB,), jnp.int32),
        pltpu.VMEM((CB,), jnp.int32),
        pltpu.VMEM((CB,), jnp.int32),
        pltpu.VMEM((CB,), jnp.int32),
        pltpu.VMEM((CB, D), jnp.float32),
        pltpu.VMEM((CB, D), jnp.float32),
        pltpu.VMEM((CB,), jnp.float32),
        pltpu.VMEM((CB,), jnp.float32),
        pltpu.VMEM_SHARED((NP, D), jnp.float32),
        pltpu.SemaphoreType.DMA,
        pltpu.SemaphoreType.DMA,
    ],
)
def _gcn_sc(xwp_hbm, src_hbm, dst_hbm, ew_hbm, zeros_hbm, out_hbm,
            srciA, dstiA, srciB, dstiB, xwvA, xwvB, ewvA, ewvB,
            acc, semA, semB):
    c = lax.axis_index("c")
    s = lax.axis_index("s")
    wid = s * NC + c

    pltpu.sync_copy(zeros_hbm.at[pl.ds(s * RPT, RPT)],
                    acc.at[pl.ds(s * RPT, RPT)])
    plsc.subcore_barrier()

    def issue(chunk, srci, dsti, xwv, ewv, sem):
        off = chunk * CB
        pltpu.sync_copy(dst_hbm.at[pl.ds(off, CB)], dsti)
        pltpu.sync_copy(src_hbm.at[pl.ds(off, CB)], srci)
        pltpu.sync_copy(ew_hbm.at[pl.ds(off, CB)], ewv)
        pltpu.async_copy(xwp_hbm.at[srci], xwv, sem)

    def process(chunk, srci, dsti, xwv, ewv, sem):
        pltpu.make_async_copy(xwp_hbm.at[srci], xwv, sem).wait()

        def e_body(jj, c2):
            evec = ewv[pl.ds(jj * 16, 16)]
            for l in range(16):
                i = jj * 16 + l
                nrm = evec[l]
                for d8 in range(8):
                    sl = pl.ds(d8 * 16, 16)
                    xwv[i, sl] = xwv[i, sl] * nrm
            return c2

        lax.fori_loop(0, CB // 16, e_body, 0)
        pltpu.sync_copy(xwv, acc.at[dsti], add=True)

    issue(wid, srciA, dstiA, xwvA, ewvA, semA)

    def t_body(tp, carry):
        c0 = (2 * tp) * NW + wid
        c1 = c0 + NW
        c2 = c1 + NW

        @pl.when(c1 < NCHUNK)
        def _():
            issue(c1, srciB, dstiB, xwvB, ewvB, semB)

        @pl.when(c0 < NCHUNK)
        def _():
            process(c0, srciA, dstiA, xwvA, ewvA, semA)

        @pl.when(c2 < NCHUNK)
        def _():
            issue(c2, srciA, dstiA, xwvA, ewvA, semA)

        @pl.when(c1 < NCHUNK)
        def _():
            process(c1, srciB, dstiB, xwvB, ewvB, semB)

        return carry

    lax.fori_loop(0, NPAIR, t_body, 0)
    plsc.subcore_barrier()
    pltpu.sync_copy(acc.at[pl.ds(s * RPT, RPT)],
                    out_hbm.at[pl.ds(c * NP + s * RPT, RPT)])


def _mm4_body(x_ref, w_ref, b_ref, q_ref, k_ref, v_ref, r_ref):
    o = (jnp.dot(x_ref[...], w_ref[...], preferred_element_type=jnp.float32)
         + b_ref[...])
    q_ref[...] = o[:, 0:D]
    k_ref[...] = o[:, D:2 * D]
    v_ref[...] = o[:, 2 * D:3 * D]
    r_ref[...] = o[:, 3 * D:4 * D]


def _mm4(x, w4, b4):
    os = jax.ShapeDtypeStruct((N, D), jnp.float32)
    return pl.pallas_call(
        _mm4_body,
        grid=(N // ROWS,),
        in_specs=[
            pl.BlockSpec((ROWS, D), lambda i: (i, 0)),
            pl.BlockSpec((D, 4 * D), lambda i: (0, 0)),
            pl.BlockSpec((1, 4 * D), lambda i: (0, 0)),
        ],
        out_specs=[pl.BlockSpec((ROWS, D), lambda i: (i, 0))] * 4,
        out_shape=[os, os, os, os],
    )(x, w4, b4.reshape(1, 4 * D))


def _softmax_finish(acc_ref, dinv_ref, r_ref):
    return (acc_ref[0] + acc_ref[1]) * dinv_ref[...] + r_ref[...]


def _epA_mm_body(acc_ref, dinv_ref, r_ref, w_ref, b_ref, o_ref):
    h = _softmax_finish(acc_ref, dinv_ref, r_ref)
    o_ref[...] = (jnp.dot(h, w_ref[...], preferred_element_type=jnp.float32)
                  + b_ref[...])


def _epA_mm(acc, deninv, r, w, b):
    kk = w.shape[1]
    return pl.pallas_call(
        _epA_mm_body,
        grid=(N // ROWS,),
        in_specs=[
            pl.BlockSpec((2, ROWS, D), lambda i: (0, i, 0)),
            pl.BlockSpec((ROWS, 1), lambda i: (i, 0)),
            pl.BlockSpec((ROWS, D), lambda i: (i, 0)),
            pl.BlockSpec((D, kk), lambda i: (0, 0)),
            pl.BlockSpec((1, kk), lambda i: (0, 0)),
        ],
        out_specs=pl.BlockSpec((ROWS, kk), lambda i: (i, 0)),
        out_shape=jax.ShapeDtypeStruct((N, kk), jnp.float32),
    )(acc.reshape(NC, NP, D)[:, :N], deninv, r, w, b.reshape(1, kk))


def _epA_mm4_body(acc_ref, dinv_ref, r_ref, w_ref, b_ref,
                  q_ref, k_ref, v_ref, r4_ref):
    h = _softmax_finish(acc_ref, dinv_ref, r_ref)
    o = (jnp.dot(h, w_ref[...], preferred_element_type=jnp.float32)
         + b_ref[...])
    q_ref[...] = o[:, 0:D]
    k_ref[...] = o[:, D:2 * D]
    v_ref[...] = o[:, 2 * D:3 * D]
    r4_ref[...] = o[:, 3 * D:4 * D]


def _epA_mm4(acc, deninv, r, w4, b4):
    os = jax.ShapeDtypeStruct((N, D), jnp.float32)
    return pl.pallas_call(
        _epA_mm4_body,
        grid=(N // ROWS,),
        in_specs=[
            pl.BlockSpec((2, ROWS, D), lambda i: (0, i, 0)),
            pl.BlockSpec((ROWS, 1), lambda i: (i, 0)),
            pl.BlockSpec((ROWS, D), lambda i: (i, 0)),
            pl.BlockSpec((D, 4 * D), lambda i: (0, 0)),
            pl.BlockSpec((1, 4 * D), lambda i: (0, 0)),
        ],
        out_specs=[pl.BlockSpec((ROWS, D), lambda i: (i, 0))] * 4,
        out_shape=[os, os, os, os],
    )(acc.reshape(NC, NP, D)[:, :N], deninv, r, w4, b4.reshape(1, 4 * D))


def _epA_mm_gcn_body(acc_ref, deninv_ref, r_ref, w_ref, dinv_ref,
                     xw_ref, xwp_ref):
    h = _softmax_finish(acc_ref, deninv_ref, r_ref)
    xw = jnp.dot(h, w_ref[...], preferred_element_type=jnp.float32)
    xw_ref[...] = xw
    xwp_ref[...] = xw * dinv_ref[...]


def _epA_mm_gcn(acc, deninv, r, w, dinv):
    os = jax.ShapeDtypeStruct((N, D), jnp.float32)
    return pl.pallas_call(
        _epA_mm_gcn_body,
        grid=(N // ROWS,),
        in_specs=[
            pl.BlockSpec((2, ROWS, D), lambda i: (0, i, 0)),
            pl.BlockSpec((ROWS, 1), lambda i: (i, 0)),
            pl.BlockSpec((ROWS, D), lambda i: (i, 0)),
            pl.BlockSpec((D, D), lambda i: (0, 0)),
            pl.BlockSpec((ROWS, 1), lambda i: (i, 0)),
        ],
        out_specs=[pl.BlockSpec((ROWS, D), lambda i: (i, 0))] * 2,
        out_shape=[os, os],
    )(acc.reshape(NC, NP, D)[:, :N], deninv, r, w, dinv)


def _epG_mm4_body(acc_ref, xw_ref, dinv_ref, invdeg_ref, gb_ref, w_ref,
                  b_ref, q_ref, k_ref, v_ref, r4_ref):
    h = ((acc_ref[0] + acc_ref[1]) * dinv_ref[...]
         + xw_ref[...] * invdeg_ref[...] + gb_ref[...])
    o = (jnp.dot(h, w_ref[...], preferred_element_type=jnp.float32)
         + b_ref[...])
    q_ref[...] = o[:, 0:D]
    k_ref[...] = o[:, D:2 * D]
    v_ref[...] = o[:, 2 * D:3 * D]
    r4_ref[...] = o[:, 3 * D:4 * D]


def _epG_mm4(acc, xw, dinv, invdeg, g_b, w4, b4):
    os = jax.ShapeDtypeStruct((N, D), jnp.float32)
    return pl.pallas_call(
        _epG_mm4_body,
        grid=(N // ROWS,),
        in_specs=[
            pl.BlockSpec((2, ROWS, D), lambda i: (0, i, 0)),
            pl.BlockSpec((ROWS, D), lambda i: (i, 0)),
            pl.BlockSpec((ROWS, 1), lambda i: (i, 0)),
            pl.BlockSpec((ROWS, 1), lambda i: (i, 0)),
            pl.BlockSpec((1, D), lambda i: (0, 0)),
            pl.BlockSpec((D, 4 * D), lambda i: (0, 0)),
            pl.BlockSpec((1, 4 * D), lambda i: (0, 0)),
        ],
        out_specs=[pl.BlockSpec((ROWS, D), lambda i: (i, 0))] * 4,
        out_shape=[os, os, os, os],
    )(acc.reshape(NC, NP, D)[:, :N], xw, dinv, invdeg, g_b.reshape(1, D), w4,
      b4.reshape(1, 4 * D))


def _deninv(den):
    d2 = den.reshape(NC, NP)[:, :N]
    return (1.0 / (d2[0] + d2[1] + 1e-16))[:, None]


def kernel(x, edge_index, edge_wt, batch,
           t1_Wq, t1_bq, t1_Wk, t1_bk, t1_Wv, t1_bv, t1_Ws, t1_bs,
           t0_Wq, t0_bq, t0_Wk, t0_bk, t0_Wv, t0_bv, t0_Ws, t0_bs,
           g_W, g_b, fc_W, fc_b):
    src, dst = edge_index[0], edge_index[1]
    zerosD = jnp.zeros((NP, D), jnp.float32)
    zeros1 = jnp.zeros((NP,), jnp.float32)
    w4_t1 = jnp.concatenate([t1_Wq, t1_Wk, t1_Wv, t1_Ws], axis=1)
    b4_t1 = jnp.concatenate([t1_bq, t1_bk, t1_bv, t1_bs])
    w4_t0 = jnp.concatenate([t0_Wq, t0_Wk, t0_Wv, t0_Ws], axis=1)
    b4_t0 = jnp.concatenate([t0_bq, t0_bk, t0_bv, t0_bs])
    fcw_pad = jnp.pad(fc_W, ((0, 0), (0, D - 1)))
    fcb_pad = jnp.pad(fc_b, (0, D - 1))

    q, k, v, r = _mm4(x, w4_t1, b4_t1)
    alpha, amaxp = _alpha_sc(q, k, src, dst)
    gmax16 = jnp.full((16,), jnp.max(amaxp), jnp.float32)
    acc1, den1, ewa1 = _aggr_ew(v, src, dst, alpha, gmax16, edge_wt,
                                zerosD, zeros1)

    e2 = ewa1.reshape(NC, NP)[:, :N]
    deg = e2[0] + e2[1] + 1.0
    dinv = lax.rsqrt(deg)[:, None]
    invdeg = (1.0 / deg)[:, None]

    xw, xwp = _epA_mm_gcn(acc1, _deninv(den1), r, g_W, dinv)
    accC = _gcn_sc(xwp, src, dst, edge_wt, zerosD)

    q, k, v, r = _epG_mm4(accC, xw, dinv, invdeg, g_b, w4_t0, b4_t0)
    alpha, amaxp = _alpha_sc(q, k, src, dst)
    gmax16 = jnp.full((16,), jnp.max(amaxp), jnp.float32)
    acc2, den2 = _aggr_plain(v, src, dst, alpha, gmax16, edge_wt,
                             zerosD, zeros1)

    q, k, v, r = _epA_mm4(acc2, _deninv(den2), r, w4_t1, b4_t1)
    alpha, amaxp = _alpha_sc(q, k, src, dst)
    gmax16 = jnp.full((16,), jnp.max(amaxp), jnp.float32)
    acc3, den3 = _aggr_plain(v, src, dst, alpha, gmax16, edge_wt,
                             zerosD, zeros1)

    return _epA_mm(acc3, _deninv(den3), r, fcw_pad, fcb_pad)[:, 0:1]


# speedup vs baseline: 20.8940x; 1.5125x over previous
import functools

import jax
import jax.numpy as jnp
from jax import lax
from jax.experimental import pallas as pl
from jax.experimental.pallas import tpu as pltpu
from jax.experimental.pallas import tpu_sc as plsc

N = 10000
E = 320000
D = 128

NC = 2
NS = 16
NW = NC * NS
CB = 128
NCHUNK = E // CB
NITER = -(-NCHUNK // NW)
NPAIR = -(-NITER // 2)
NQUAD = -(-NITER // 4)
NP = 10240
RPT = NP // NS
SCALE = 1.0 / float(D) ** 0.5
ROWS = 1000


def _mesh():
    return plsc.VectorSubcoreMesh(core_axis_name="c", subcore_axis_name="s")


@functools.partial(
    pl.kernel,
    out_type=(jax.ShapeDtypeStruct((E,), jnp.float32),
              jax.ShapeDtypeStruct((NW * 16,), jnp.float32)),
    mesh=_mesh(),
    scratch_types=(
        [pltpu.VMEM((CB,), jnp.int32)] * 8
        + [pltpu.VMEM((CB, D), jnp.float32)] * 4
        + [pltpu.VMEM((CB,), jnp.float32)] * 4
        + [pltpu.VMEM((16,), jnp.float32)]
        + [pltpu.SemaphoreType.DMA] * 12
    ),
)
def _alpha_sc(q_hbm, k_hbm, src_hbm, dst_hbm, alpha_hbm, amax_hbm,
              srci0, srci1, srci2, srci3, dsti0, dsti1, dsti2, dsti3,
              qvA, kvA, qvB, kvB, av0, av1, av2, av3, mxv,
              semI0, semI1, semI2, semI3,
              semA1, semA2, semB1, semB2,
              semS0, semS1, semS2, semS3):
    c = lax.axis_index("c")
    s = lax.axis_index("s")
    wid = s * NC + c
    mxv[...] = jnp.full((16,), -3e38, jnp.float32)
    i16 = lax.iota(jnp.int32, 16)
    sh8 = (i16 + 8) & 15
    sh4 = (i16 + 4) & 15
    sh2 = (i16 + 2) & 15
    sh1 = (i16 + 1) & 15

    srcis = [srci0, srci1, srci2, srci3]
    dstis = [dsti0, dsti1, dsti2, dsti3]
    avs = [av0, av1, av2, av3]
    semIs = [semI0, semI1, semI2, semI3]
    semSs = [semS0, semS1, semS2, semS3]
    gsets = [(qvA, kvA, semA1, semA2), (qvB, kvB, semB1, semB2)]

    def pre(chunk, j):
        off = chunk * CB
        pltpu.async_copy(dst_hbm.at[pl.ds(off, CB)], dstis[j], semIs[j])
        pltpu.async_copy(src_hbm.at[pl.ds(off, CB)], srcis[j], semIs[j])

    def gath(chunk, j, g):
        off = chunk * CB
        qv, kv, s1, s2 = gsets[g]
        pltpu.make_async_copy(dst_hbm.at[pl.ds(off, CB)], dstis[j],
                              semIs[j]).wait()
        pltpu.make_async_copy(src_hbm.at[pl.ds(off, CB)], srcis[j],
                              semIs[j]).wait()
        pltpu.async_copy(q_hbm.at[dstis[j]], qv, s1)
        pltpu.async_copy(k_hbm.at[srcis[j]], kv, s2)

    def process(chunk, j, g):
        qv, kv, s1, s2 = gsets[g]
        av = avs[j]
        pltpu.make_async_copy(q_hbm.at[dstis[j]], qv, s1).wait()
        pltpu.make_async_copy(k_hbm.at[srcis[j]], kv, s2).wait()

        @pl.when(chunk >= 4 * NW)
        def _():
            pltpu.make_async_copy(
                av, alpha_hbm.at[pl.ds((chunk - 4 * NW) * CB, CB)],
                semSs[j]).wait()

        def e_body(jj, c2):
            vec = jnp.zeros((16,), jnp.float32)
            for l in range(16):
                i = jj * 16 + l
                acc = qv[i, pl.ds(0, 16)] * kv[i, pl.ds(0, 16)]
                for d8 in range(1, 8):
                    acc = acc + (qv[i, pl.ds(d8 * 16, 16)]
                                 * kv[i, pl.ds(d8 * 16, 16)])
                acc = acc + acc[sh8]
                acc = acc + acc[sh4]
                acc = acc + acc[sh2]
                acc = acc + acc[sh1]
                vec = jnp.where(i16 == l, acc * SCALE, vec)
            av[pl.ds(jj * 16, 16)] = vec
            mxv[...] = jnp.maximum(mxv[...], vec)
            return c2

        lax.fori_loop(0, CB // 16, e_body, 0)
        pltpu.async_copy(av, alpha_hbm.at[pl.ds(chunk * CB, CB)], semSs[j])

    for j in range(4):
        pre(wid + j * NW, j)
    gath(wid, 0, 0)

    def t_body(tq, carry):
        cs = [(4 * tq + j) * NW + wid for j in range(4)]
        ds = [cc + 4 * NW for cc in cs]

        def when_do(cond_chunk, fn, *a):
            @pl.when(cond_chunk < NCHUNK)
            def _():
                fn(*a)

        when_do(cs[1], gath, cs[1], 1, 1)
        when_do(cs[0], process, cs[0], 0, 0)
        when_do(ds[0], pre, ds[0], 0)
        when_do(cs[2], gath, cs[2], 2, 0)
        when_do(cs[1], process, cs[1], 1, 1)
        when_do(ds[1], pre, ds[1], 1)
        when_do(cs[3], gath, cs[3], 3, 1)
        when_do(cs[2], process, cs[2], 2, 0)
        when_do(ds[2], pre, ds[2], 2)
        when_do(ds[0], gath, ds[0], 0, 0)
        when_do(cs[3], process, cs[3], 3, 1)
        when_do(ds[3], pre, ds[3], 3)
        return carry

    lax.fori_loop(0, NQUAD, t_body, 0)
    for j in range(4):
        pltpu.make_async_copy(avs[j], alpha_hbm.at[pl.ds(wid * CB, CB)],
                              semSs[j]).wait()
    pltpu.sync_copy(mxv, amax_hbm.at[pl.ds(wid * 16, 16)])


def _make_aggr(with_ew):
    scratch = (
        [pltpu.VMEM((CB,), jnp.int32)] * 8
        + [pltpu.VMEM((CB, D), jnp.float32)] * 2
        + [pltpu.VMEM((CB,), jnp.float32)] * 8
        + [pltpu.VMEM((CB,), jnp.float32),
           pltpu.VMEM((16,), jnp.float32),
           pltpu.VMEM_SHARED((NP, D), jnp.float32),
           pltpu.VMEM_SHARED((NP,), jnp.float32),
           pltpu.VMEM_SHARED((NP,), jnp.float32)]
        + [pltpu.SemaphoreType.DMA] * 6
    )
    outs = [jax.ShapeDtypeStruct((NC * NP, D), jnp.float32),
            jax.ShapeDtypeStruct((NC * NP,), jnp.float32)]
    if with_ew:
        outs.append(jax.ShapeDtypeStruct((NC * NP,), jnp.float32))

    @functools.partial(
        pl.kernel,
        out_type=tuple(outs),
        mesh=_mesh(),
        scratch_types=scratch,
    )
    def _aggr_sc(v_hbm, src_hbm, dst_hbm, alpha_hbm, gmax_hbm, ew_hbm,
                 zerosd_hbm, zeros1_hbm, *refs):
        if with_ew:
            (out_hbm, den_hbm, ewa_hbm,
             srci0, srci1, srci2, srci3, dsti0, dsti1, dsti2, dsti3,
             vvA, vvB, av0, av1, av2, av3, ewv0, ewv1, ewv2, ewv3,
             exv, gv, acc, den, ewa,
             semI0, semI1, semI2, semI3, semA, semB) = refs
        else:
            (out_hbm, den_hbm,
             srci0, srci1, srci2, srci3, dsti0, dsti1, dsti2, dsti3,
             vvA, vvB, av0, av1, av2, av3, ewv0, ewv1, ewv2, ewv3,
             exv, gv, acc, den, ewa,
             semI0, semI1, semI2, semI3, semA, semB) = refs
        c = lax.axis_index("c")
        s = lax.axis_index("s")
        wid = s * NC + c

        pltpu.sync_copy(zerosd_hbm.at[pl.ds(s * RPT, RPT)],
                        acc.at[pl.ds(s * RPT, RPT)])
        pltpu.sync_copy(zeros1_hbm.at[pl.ds(s * RPT, RPT)],
                        den.at[pl.ds(s * RPT, RPT)])
        if with_ew:
            pltpu.sync_copy(zeros1_hbm.at[pl.ds(s * RPT, RPT)],
                            ewa.at[pl.ds(s * RPT, RPT)])
        pltpu.sync_copy(gmax_hbm, gv)
        plsc.subcore_barrier()
        gvec = gv[...]

        srcis = [srci0, srci1, srci2, srci3]
        dstis = [dsti0, dsti1, dsti2, dsti3]
        avs = [av0, av1, av2, av3]
        ewvs = [ewv0, ewv1, ewv2, ewv3]
        semIs = [semI0, semI1, semI2, semI3]
        gsets = [(vvA, semA), (vvB, semB)]

        def pre(chunk, j):
            off = chunk * CB
            pltpu.async_copy(dst_hbm.at[pl.ds(off, CB)], dstis[j], semIs[j])
            pltpu.async_copy(src_hbm.at[pl.ds(off, CB)], srcis[j], semIs[j])
            pltpu.async_copy(alpha_hbm.at[pl.ds(off, CB)], avs[j], semIs[j])
            if with_ew:
                pltpu.async_copy(ew_hbm.at[pl.ds(off, CB)], ewvs[j],
                                 semIs[j])

        def gath(chunk, j, g):
            off = chunk * CB
            vv, sem = gsets[g]
            pltpu.make_async_copy(dst_hbm.at[pl.ds(off, CB)], dstis[j],
                                  semIs[j]).wait()
            pltpu.make_async_copy(src_hbm.at[pl.ds(off, CB)], srcis[j],
                                  semIs[j]).wait()
            pltpu.async_copy(v_hbm.at[srcis[j]], vv, sem)

        def process(chunk, j, g):
            off = chunk * CB
            vv, sem = gsets[g]
            dsti = dstis[j]
            pltpu.make_async_copy(alpha_hbm.at[pl.ds(off, CB)], avs[j],
                                  semIs[j]).wait()
            if with_ew:
                pltpu.make_async_copy(ew_hbm.at[pl.ds(off, CB)], ewvs[j],
                                      semIs[j]).wait()
                pltpu.sync_copy(ewvs[j], ewa.at[dsti], add=True)
            av = avs[j]

            def x_body(jx, c2):
                sl = pl.ds(jx * 16, 16)
                exv[sl] = jnp.exp(av[sl] - gvec)
                return c2

            lax.fori_loop(0, CB // 16, x_body, 0)
            pltpu.sync_copy(exv, den.at[dsti], add=True)
            pltpu.make_async_copy(v_hbm.at[srcis[j]], vv, sem).wait()

            def e_body(jj, c2):
                exvec = exv[pl.ds(jj * 16, 16)]
                for l in range(16):
                    i = jj * 16 + l
                    e = exvec[l]
                    for d8 in range(8):
                        sl = pl.ds(d8 * 16, 16)
                        vv[i, sl] = vv[i, sl] * e
                return c2

            lax.fori_loop(0, CB // 16, e_body, 0)
            pltpu.sync_copy(vv, acc.at[dsti], add=True)

        for j in range(4):
            pre(wid + j * NW, j)
        gath(wid, 0, 0)

        def t_body(tq, carry):
            cs = [(4 * tq + j) * NW + wid for j in range(4)]
            dss = [cc + 4 * NW for cc in cs]

            def when_do(cond_chunk, fn, *a):
                @pl.when(cond_chunk < NCHUNK)
                def _():
                    fn(*a)

            when_do(cs[1], gath, cs[1], 1, 1)
            when_do(cs[0], process, cs[0], 0, 0)
            when_do(dss[0], pre, dss[0], 0)
            when_do(cs[2], gath, cs[2], 2, 0)
            when_do(cs[1], process, cs[1], 1, 1)
            when_do(dss[1], pre, dss[1], 1)
            when_do(cs[3], gath, cs[3], 3, 1)
            when_do(cs[2], process, cs[2], 2, 0)
            when_do(dss[2], pre, dss[2], 2)
            when_do(dss[0], gath, dss[0], 0, 0)
            when_do(cs[3], process, cs[3], 3, 1)
            when_do(dss[3], pre, dss[3], 3)
            return carry

        lax.fori_loop(0, NQUAD, t_body, 0)
        plsc.subcore_barrier()
        pltpu.sync_copy(acc.at[pl.ds(s * RPT, RPT)],
                        out_hbm.at[pl.ds(c * NP + s * RPT, RPT)])
        pltpu.sync_copy(den.at[pl.ds(s * RPT, RPT)],
                        den_hbm.at[pl.ds(c * NP + s * RPT, RPT)])
        if with_ew:
            pltpu.sync_copy(ewa.at[pl.ds(s * RPT, RPT)],
                            ewa_hbm.at[pl.ds(c * NP + s * RPT, RPT)])

    return _aggr_sc


_aggr_ew = _make_aggr(True)
_aggr_plain = _make_aggr(False)


@functools.partial(
    pl.kernel,
    out_type=jax.ShapeDtypeStruct((NC * NP, D), jnp.float32),
    mesh=_mesh(),
    scratch_types=(
        [pltpu.VMEM((CB,), jnp.int32)] * 8
        + [pltpu.VMEM((CB, D), jnp.float32)] * 2
        + [pltpu.VMEM((CB,), jnp.float32)] * 4
        + [pltpu.VMEM_SHARED((NP, D), jnp.float32)]
        + [pltpu.SemaphoreType.DMA] * 6
    ),
)
def _gcn_sc(xwp_hbm, src_hbm, dst_hbm, ew_hbm, zeros_hbm, out_hbm,
            srci0, srci1, srci2, srci3, dsti0, dsti1, dsti2, dsti3,
            xwvA, xwvB, ewv0, ewv1, ewv2, ewv3, acc,
            semI0, semI1, semI2, semI3, semA, semB):
    c = lax.axis_index("c")
    s = lax.axis_index("s")
    wid = s * NC + c

    pltpu.sync_copy(zeros_hbm.at[pl.ds(s * RPT, RPT)],
                    acc.at[pl.ds(s * RPT, RPT)])
    plsc.subcore_barrier()

    srcis = [srci0, srci1, srci2, srci3]
    dstis = [dsti0, dsti1, dsti2, dsti3]
    ewvs = [ewv0, ewv1, ewv2, ewv3]
    semIs = [semI0, semI1, semI2, semI3]
    gsets = [(xwvA, semA), (xwvB, semB)]

    def pre(chunk, j):
        off = chunk * CB
        pltpu.async_copy(dst_hbm.at[pl.ds(off, CB)], dstis[j], semIs[j])
        pltpu.async_copy(src_hbm.at[pl.ds(off, CB)], srcis[j], semIs[j])
        pltpu.async_copy(ew_hbm.at[pl.ds(off, CB)], ewvs[j], semIs[j])

    def gath(chunk, j, g):
        off = chunk * CB
        xwv, sem = gsets[g]
        pltpu.make_async_copy(dst_hbm.at[pl.ds(off, CB)], dstis[j],
                              semIs[j]).wait()
        pltpu.make_async_copy(src_hbm.at[pl.ds(off, CB)], srcis[j],
                              semIs[j]).wait()
        pltpu.async_copy(xwp_hbm.at[srcis[j]], xwv, sem)

    def process(chunk, j, g):
        off = chunk * CB
        xwv, sem = gsets[g]
        ewv = ewvs[j]
        pltpu.make_async_copy(ew_hbm.at[pl.ds(off, CB)], ewv,
                              semIs[j]).wait()
        pltpu.make_async_copy(xwp_hbm.at[srcis[j]], xwv, sem).wait()

        def e_body(jj, c2):
            evec = ewv[pl.ds(jj * 16, 16)]
            for l in range(16):
                i = jj * 16 + l
                nrm = evec[l]
                for d8 in range(8):
                    sl = pl.ds(d8 * 16, 16)
                    xwv[i, sl] = xwv[i, sl] * nrm
            return c2

        lax.fori_loop(0, CB // 16, e_body, 0)
        pltpu.sync_copy(xwv, acc.at[dstis[j]], add=True)

    for j in range(4):
        pre(wid + j * NW, j)
    gath(wid, 0, 0)

    def t_body(tq, carry):
        cs = [(4 * tq + j) * NW + wid for j in range(4)]
        dss = [cc + 4 * NW for cc in cs]

        def when_do(cond_chunk, fn, *a):
            @pl.when(cond_chunk < NCHUNK)
            def _():
                fn(*a)

        when_do(cs[1], gath, cs[1], 1, 1)
        when_do(cs[0], process, cs[0], 0, 0)
        when_do(dss[0], pre, dss[0], 0)
        when_do(cs[2], gath, cs[2], 2, 0)
        when_do(cs[1], process, cs[1], 1, 1)
        when_do(dss[1], pre, dss[1], 1)
        when_do(cs[3], gath, cs[3], 3, 1)
        when_do(cs[2], process, cs[2], 2, 0)
        when_do(dss[2], pre, dss[2], 2)
        when_do(dss[0], gath, dss[0], 0, 0)
        when_do(cs[3], process, cs[3], 3, 1)
        when_do(dss[3], pre, dss[3], 3)
        return carry

    lax.fori_loop(0, NQUAD, t_body, 0)
    plsc.subcore_barrier()
    pltpu.sync_copy(acc.at[pl.ds(s * RPT, RPT)],
                    out_hbm.at[pl.ds(c * NP + s * RPT, RPT)])


def _mm4_body(x_ref, w_ref, b_ref, q_ref, k_ref, v_ref, r_ref):
    o = (jnp.dot(x_ref[...], w_ref[...], preferred_element_type=jnp.float32)
         + b_ref[...])
    q_ref[...] = o[:, 0:D]
    k_ref[...] = o[:, D:2 * D]
    v_ref[...] = o[:, 2 * D:3 * D]
    r_ref[...] = o[:, 3 * D:4 * D]


def _mm4(x, w4, b4):
    os = jax.ShapeDtypeStruct((N, D), jnp.float32)
    return pl.pallas_call(
        _mm4_body,
        grid=(N // ROWS,),
        in_specs=[
            pl.BlockSpec((ROWS, D), lambda i: (i, 0)),
            pl.BlockSpec((D, 4 * D), lambda i: (0, 0)),
            pl.BlockSpec((1, 4 * D), lambda i: (0, 0)),
        ],
        out_specs=[pl.BlockSpec((ROWS, D), lambda i: (i, 0))] * 4,
        out_shape=[os, os, os, os],
    )(x, w4, b4.reshape(1, 4 * D))


def _softmax_finish(acc_ref, dinv_ref, r_ref):
    return (acc_ref[0] + acc_ref[1]) * dinv_ref[...] + r_ref[...]


def _epA_mm_body(acc_ref, dinv_ref, r_ref, w_ref, b_ref, o_ref):
    h = _softmax_finish(acc_ref, dinv_ref, r_ref)
    o_ref[...] = (jnp.dot(h, w_ref[...], preferred_element_type=jnp.float32)
                  + b_ref[...])


def _epA_mm(acc, deninv, r, w, b):
    kk = w.shape[1]
    return pl.pallas_call(
        _epA_mm_body,
        grid=(N // ROWS,),
        in_specs=[
            pl.BlockSpec((2, ROWS, D), lambda i: (0, i, 0)),
            pl.BlockSpec((ROWS, 1), lambda i: (i, 0)),
            pl.BlockSpec((ROWS, D), lambda i: (i, 0)),
            pl.BlockSpec((D, kk), lambda i: (0, 0)),
            pl.BlockSpec((1, kk), lambda i: (0, 0)),
        ],
        out_specs=pl.BlockSpec((ROWS, kk), lambda i: (i, 0)),
        out_shape=jax.ShapeDtypeStruct((N, kk), jnp.float32),
    )(acc.reshape(NC, NP, D)[:, :N], deninv, r, w, b.reshape(1, kk))


def _epA_mm4_body(acc_ref, dinv_ref, r_ref, w_ref, b_ref,
                  q_ref, k_ref, v_ref, r4_ref):
    h = _softmax_finish(acc_ref, dinv_ref, r_ref)
    o = (jnp.dot(h, w_ref[...], preferred_element_type=jnp.float32)
         + b_ref[...])
    q_ref[...] = o[:, 0:D]
    k_ref[...] = o[:, D:2 * D]
    v_ref[...] = o[:, 2 * D:3 * D]
    r4_ref[...] = o[:, 3 * D:4 * D]


def _epA_mm4(acc, deninv, r, w4, b4):
    os = jax.ShapeDtypeStruct((N, D), jnp.float32)
    return pl.pallas_call(
        _epA_mm4_body,
        grid=(N // ROWS,),
        in_specs=[
            pl.BlockSpec((2, ROWS, D), lambda i: (0, i, 0)),
            pl.BlockSpec((ROWS, 1), lambda i: (i, 0)),
            pl.BlockSpec((ROWS, D), lambda i: (i, 0)),
            pl.BlockSpec((D, 4 * D), lambda i: (0, 0)),
            pl.BlockSpec((1, 4 * D), lambda i: (0, 0)),
        ],
        out_specs=[pl.BlockSpec((ROWS, D), lambda i: (i, 0))] * 4,
        out_shape=[os, os, os, os],
    )(acc.reshape(NC, NP, D)[:, :N], deninv, r, w4, b4.reshape(1, 4 * D))


def _epA_mm_gcn_body(acc_ref, deninv_ref, r_ref, w_ref, dinv_ref,
                     xw_ref, xwp_ref):
    h = _softmax_finish(acc_ref, deninv_ref, r_ref)
    xw = jnp.dot(h, w_ref[...], preferred_element_type=jnp.float32)
    xw_ref[...] = xw
    xwp_ref[...] = xw * dinv_ref[...]


def _epA_mm_gcn(acc, deninv, r, w, dinv):
    os = jax.ShapeDtypeStruct((N, D), jnp.float32)
    return pl.pallas_call(
        _epA_mm_gcn_body,
        grid=(N // ROWS,),
        in_specs=[
            pl.BlockSpec((2, ROWS, D), lambda i: (0, i, 0)),
            pl.BlockSpec((ROWS, 1), lambda i: (i, 0)),
            pl.BlockSpec((ROWS, D), lambda i: (i, 0)),
            pl.BlockSpec((D, D), lambda i: (0, 0)),
            pl.BlockSpec((ROWS, 1), lambda i: (i, 0)),
        ],
        out_specs=[pl.BlockSpec((ROWS, D), lambda i: (i, 0))] * 2,
        out_shape=[os, os],
    )(acc.reshape(NC, NP, D)[:, :N], deninv, r, w, dinv)


def _epG_mm4_body(acc_ref, xw_ref, dinv_ref, invdeg_ref, gb_ref, w_ref,
                  b_ref, q_ref, k_ref, v_ref, r4_ref):
    h = ((acc_ref[0] + acc_ref[1]) * dinv_ref[...]
         + xw_ref[...] * invdeg_ref[...] + gb_ref[...])
    o = (jnp.dot(h, w_ref[...], preferred_element_type=jnp.float32)
         + b_ref[...])
    q_ref[...] = o[:, 0:D]
    k_ref[...] = o[:, D:2 * D]
    v_ref[...] = o[:, 2 * D:3 * D]
    r4_ref[...] = o[:, 3 * D:4 * D]


def _epG_mm4(acc, xw, dinv, invdeg, g_b, w4, b4):
    os = jax.ShapeDtypeStruct((N, D), jnp.float32)
    return pl.pallas_call(
        _epG_mm4_body,
        grid=(N // ROWS,),
        in_specs=[
            pl.BlockSpec((2, ROWS, D), lambda i: (0, i, 0)),
            pl.BlockSpec((ROWS, D), lambda i: (i, 0)),
            pl.BlockSpec((ROWS, 1), lambda i: (i, 0)),
            pl.BlockSpec((ROWS, 1), lambda i: (i, 0)),
            pl.BlockSpec((1, D), lambda i: (0, 0)),
            pl.BlockSpec((D, 4 * D), lambda i: (0, 0)),
            pl.BlockSpec((1, 4 * D), lambda i: (0, 0)),
        ],
        out_specs=[pl.BlockSpec((ROWS, D), lambda i: (i, 0))] * 4,
        out_shape=[os, os, os, os],
    )(acc.reshape(NC, NP, D)[:, :N], xw, dinv, invdeg, g_b.reshape(1, D), w4,
      b4.reshape(1, 4 * D))


def _deninv(den):
    d2 = den.reshape(NC, NP)[:, :N]
    return (1.0 / (d2[0] + d2[1] + 1e-16))[:, None]


def kernel(x, edge_index, edge_wt, batch,
           t1_Wq, t1_bq, t1_Wk, t1_bk, t1_Wv, t1_bv, t1_Ws, t1_bs,
           t0_Wq, t0_bq, t0_Wk, t0_bk, t0_Wv, t0_bv, t0_Ws, t0_bs,
           g_W, g_b, fc_W, fc_b):
    src, dst = edge_index[0], edge_index[1]
    zerosD = jnp.zeros((NP, D), jnp.float32)
    zeros1 = jnp.zeros((NP,), jnp.float32)
    w4_t1 = jnp.concatenate([t1_Wq, t1_Wk, t1_Wv, t1_Ws], axis=1)
    b4_t1 = jnp.concatenate([t1_bq, t1_bk, t1_bv, t1_bs])
    w4_t0 = jnp.concatenate([t0_Wq, t0_Wk, t0_Wv, t0_Ws], axis=1)
    b4_t0 = jnp.concatenate([t0_bq, t0_bk, t0_bv, t0_bs])
    fcw_pad = jnp.pad(fc_W, ((0, 0), (0, D - 1)))
    fcb_pad = jnp.pad(fc_b, (0, D - 1))

    q, k, v, r = _mm4(x, w4_t1, b4_t1)
    alpha, amaxp = _alpha_sc(q, k, src, dst)
    gmax16 = jnp.full((16,), jnp.max(amaxp), jnp.float32)
    acc1, den1, ewa1 = _aggr_ew(v, src, dst, alpha, gmax16, edge_wt,
                                zerosD, zeros1)

    e2 = ewa1.reshape(NC, NP)[:, :N]
    deg = e2[0] + e2[1] + 1.0
    dinv = lax.rsqrt(deg)[:, None]
    invdeg = (1.0 / deg)[:, None]

    xw, xwp = _epA_mm_gcn(acc1, _deninv(den1), r, g_W, dinv)
    accC = _gcn_sc(xwp, src, dst, edge_wt, zerosD)

    q, k, v, r = _epG_mm4(accC, xw, dinv, invdeg, g_b, w4_t0, b4_t0)
    alpha, amaxp = _alpha_sc(q, k, src, dst)
    gmax16 = jnp.full((16,), jnp.max(amaxp), jnp.float32)
    acc2, den2 = _aggr_plain(v, src, dst, alpha, gmax16, edge_wt,
                             zerosD, zeros1)

    q, k, v, r = _epA_mm4(acc2, _deninv(den2), r, w4_t1, b4_t1)
    alpha, amaxp = _alpha_sc(q, k, src, dst)
    gmax16 = jnp.full((16,), jnp.max(amaxp), jnp.float32)
    acc3, den3 = _aggr_plain(v, src, dst, alpha, gmax16, edge_wt,
                             zerosD, zeros1)

    return _epA_mm(acc3, _deninv(den3), r, fcw_pad, fcb_pad)[:, 0:1]
